# Initial kernel scaffold; baseline (speedup 1.0000x reference)
import jax
import jax.numpy as jnp
from jax.experimental import pallas as pl


def kernel(x, edge_index, W1, b1, W2, b2):
    raise NotImplementedError("write your pallas kernel here")


# speedup vs baseline: 3.0113x; 3.0113x over previous
import functools

import jax
import jax.numpy as jnp
from jax import lax
from jax.experimental import pallas as pl
from jax.experimental.pallas import tpu as pltpu
from jax.experimental.pallas import tpu_sc as plsc

N_NODES = 10000
D = 128
NC = 2
NS = 16
NW = NC * NS
CHUNK = 128
CPW = 80
EPW = CHUNK * CPW
E_PAD = NW * EPW
ACC_ROWS = 10240
ROWS_PER_TILE = ACC_ROWS // NS
PAD_DST = N_NODES

BM = 1000


def _seg_sum_sc(h, src3, dst3):
    mesh = plsc.VectorSubcoreMesh(core_axis_name="c", subcore_axis_name="s")

    @functools.partial(
        pl.kernel,
        out_type=jax.ShapeDtypeStruct((NC, ACC_ROWS, D), jnp.float32),
        mesh=mesh,
        scratch_types=[
            pltpu.VMEM((CPW, CHUNK), jnp.int32),
            pltpu.VMEM((CPW, CHUNK), jnp.int32),
            pltpu.VMEM((CHUNK, D), jnp.float32),
            pltpu.VMEM_SHARED((ACC_ROWS, D), jnp.float32),
            pltpu.SemaphoreType.DMA,
        ],
    )
    def k(h_hbm, src_hbm, dst_hbm, out_hbm, src_v, dst_v, msg0, acc, sem0):
        cid = lax.axis_index("c")
        sid = lax.axis_index("s")
        wid = cid * NS + sid

        pltpu.sync_copy(src_hbm.at[wid], src_v)
        pltpu.sync_copy(dst_hbm.at[wid], dst_v)

        zero = jnp.zeros((16,), jnp.float32)

        def zrow(i, carry):
            for j in range(D // 16):
                msg0[i, pl.ds(j * 16, 16)] = zero
            return carry

        lax.fori_loop(0, CHUNK, zrow, 0)
        base = sid * ROWS_PER_TILE
        for kk in range(ROWS_PER_TILE // CHUNK):
            pltpu.sync_copy(msg0, acc.at[pl.ds(base + kk * CHUNK, CHUNK)])
        plsc.subcore_barrier()

        def body(c, carry):
            pltpu.async_copy(h_hbm.at[src_v.at[c]], msg0, sem0).wait()
            pltpu.sync_copy(msg0, acc.at[dst_v.at[c]], add=True)
            return carry

        lax.fori_loop(0, CPW, body, 0)
        plsc.subcore_barrier()

        for kk in range(ROWS_PER_TILE // CHUNK):
            r = base + kk * CHUNK
            pltpu.sync_copy(acc.at[pl.ds(r, CHUNK)], msg0)
            pltpu.sync_copy(msg0, out_hbm.at[cid].at[pl.ds(r, CHUNK)])

    return k(h, src3, dst3)


def _mm(x, W):
    M = x.shape[0]

    def kfn(x_ref, w_ref, o_ref):
        o_ref[...] = jnp.dot(x_ref[...], w_ref[...],
                             preferred_element_type=jnp.float32)

    return pl.pallas_call(
        kfn,
        grid=(M // BM,),
        in_specs=[pl.BlockSpec((BM, D), lambda i: (i, 0)),
                  pl.BlockSpec((D, D), lambda i: (0, 0))],
        out_specs=pl.BlockSpec((BM, D), lambda i: (i, 0)),
        out_shape=jax.ShapeDtypeStruct((M, D), jnp.float32),
    )(x, W)


def _comb_mm(acc, b2d, W):
    def kfn(a0_ref, a1_ref, b_ref, w_ref, o_ref):
        h = jnp.maximum(a0_ref[0] + a1_ref[0] + b_ref[...], 0.0)
        o_ref[...] = jnp.dot(h, w_ref[...],
                             preferred_element_type=jnp.float32)

    return pl.pallas_call(
        kfn,
        grid=(N_NODES // BM,),
        in_specs=[pl.BlockSpec((1, BM, D), lambda i: (0, i, 0)),
                  pl.BlockSpec((1, BM, D), lambda i: (1, i, 0)),
                  pl.BlockSpec((1, D), lambda i: (0, 0)),
                  pl.BlockSpec((D, D), lambda i: (0, 0))],
        out_specs=pl.BlockSpec((BM, D), lambda i: (i, 0)),
        out_shape=jax.ShapeDtypeStruct((N_NODES, D), jnp.float32),
    )(acc, acc, b2d, W)


def _comb(acc, b2d):
    def kfn(a0_ref, a1_ref, b_ref, o_ref):
        o_ref[...] = jnp.maximum(a0_ref[0] + a1_ref[0] + b_ref[...], 0.0)

    return pl.pallas_call(
        kfn,
        grid=(N_NODES // BM,),
        in_specs=[pl.BlockSpec((1, BM, D), lambda i: (0, i, 0)),
                  pl.BlockSpec((1, BM, D), lambda i: (1, i, 0)),
                  pl.BlockSpec((1, D), lambda i: (0, 0))],
        out_specs=pl.BlockSpec((BM, D), lambda i: (i, 0)),
        out_shape=jax.ShapeDtypeStruct((N_NODES, D), jnp.float32),
    )(acc, acc, b2d)


def kernel(x, edge_index, W1, b1, W2, b2):
    src = edge_index[0].astype(jnp.int32)
    dst = edge_index[1].astype(jnp.int32)
    n_edges = src.shape[0]
    pad = E_PAD - n_edges
    src3 = jnp.concatenate(
        [src, jnp.zeros((pad,), jnp.int32)]).reshape(NW, CPW, CHUNK)
    dst3 = jnp.concatenate(
        [dst, jnp.full((pad,), PAD_DST, jnp.int32)]).reshape(NW, CPW, CHUNK)
    b1r = b1.reshape(1, D)
    b2r = b2.reshape(1, D)

    h1 = _mm(x, W1)
    acc1 = _seg_sum_sc(h1, src3, dst3)
    h2 = _comb_mm(acc1, b1r, W2)
    acc2 = _seg_sum_sc(h2, src3, dst3)
    return _comb(acc2, b2r)


# speedup vs baseline: 3.3915x; 1.1262x over previous
import functools

import jax
import jax.numpy as jnp
from jax import lax
from jax.experimental import pallas as pl
from jax.experimental.pallas import tpu as pltpu
from jax.experimental.pallas import tpu_sc as plsc

N_NODES = 10000
D = 128
NC = 2
NS = 16
NW = NC * NS
CHUNK = 128
CPW = 80
EPW = CHUNK * CPW
E_PAD = NW * EPW
ACC_ROWS = 10240
ROWS_PER_TILE = ACC_ROWS // NS
PAD_DST = N_NODES

BM = 1000


def _seg_sum_sc(h, packed3):
    mesh = plsc.VectorSubcoreMesh(core_axis_name="c", subcore_axis_name="s")

    @functools.partial(
        pl.kernel,
        out_type=jax.ShapeDtypeStruct((NC, ACC_ROWS, D), jnp.float32),
        mesh=mesh,
        scratch_types=[
            pltpu.VMEM((CPW, CHUNK), jnp.int32),
            pltpu.VMEM((CHUNK,), jnp.int32),
            pltpu.VMEM((CHUNK,), jnp.int32),
            pltpu.VMEM((CHUNK,), jnp.int32),
            pltpu.VMEM((CHUNK,), jnp.int32),
            pltpu.VMEM((CHUNK, D), jnp.float32),
            pltpu.VMEM((CHUNK, D), jnp.float32),
            pltpu.VMEM_SHARED((ACC_ROWS, D), jnp.float32),
            pltpu.SemaphoreType.DMA,
            pltpu.SemaphoreType.DMA,
        ],
    )
    def k(h_hbm, packed_hbm, out_hbm, packed_v, sbuf0, sbuf1, dbuf0, dbuf1,
          msg0, msg1, acc, sem0, sem1):
        cid = lax.axis_index("c")
        sid = lax.axis_index("s")
        wid = cid * NS + sid

        pltpu.sync_copy(packed_hbm.at[wid], packed_v)

        zero = jnp.zeros((16,), jnp.float32)

        def zrow(i, carry):
            for j in range(D // 16):
                msg0[i, pl.ds(j * 16, 16)] = zero
            return carry

        lax.fori_loop(0, CHUNK, zrow, 0)
        base = sid * ROWS_PER_TILE
        for kk in range(ROWS_PER_TILE // CHUNK):
            pltpu.sync_copy(msg0, acc.at[pl.ds(base + kk * CHUNK, CHUNK)])
        plsc.subcore_barrier()

        def unpack(c, sbuf, dbuf):
            for j in range(CHUNK // 16):
                v = packed_v[c, pl.ds(j * 16, 16)]
                sbuf[pl.ds(j * 16, 16)] = lax.bitwise_and(v, 0xFFFF)
                dbuf[pl.ds(j * 16, 16)] = lax.shift_right_logical(v, 16)

        unpack(0, sbuf0, dbuf0)
        pltpu.async_copy(h_hbm.at[sbuf0], msg0, sem0)
        unpack(1, sbuf1, dbuf1)
        pltpu.async_copy(h_hbm.at[sbuf1], msg1, sem1)

        def body(i, carry):
            c = i * 2
            pltpu.make_async_copy(h_hbm.at[sbuf0], msg0, sem0).wait()
            pltpu.sync_copy(msg0, acc.at[dbuf0], add=True)
            unpack(jnp.minimum(c + 2, CPW - 1), sbuf0, dbuf0)
            pltpu.async_copy(h_hbm.at[sbuf0], msg0, sem0)
            pltpu.make_async_copy(h_hbm.at[sbuf1], msg1, sem1).wait()
            pltpu.sync_copy(msg1, acc.at[dbuf1], add=True)
            unpack(jnp.minimum(c + 3, CPW - 1), sbuf1, dbuf1)
            pltpu.async_copy(h_hbm.at[sbuf1], msg1, sem1)
            return carry

        lax.fori_loop(0, CPW // 2, body, 0)
        pltpu.make_async_copy(h_hbm.at[sbuf0], msg0, sem0).wait()
        pltpu.make_async_copy(h_hbm.at[sbuf1], msg1, sem1).wait()
        plsc.subcore_barrier()

        for kk in range(ROWS_PER_TILE // CHUNK):
            r = base + kk * CHUNK
            pltpu.sync_copy(acc.at[pl.ds(r, CHUNK)], msg0)
            pltpu.sync_copy(msg0, out_hbm.at[cid].at[pl.ds(r, CHUNK)])

    return k(h, packed3)


def _mm(x, W):
    M = x.shape[0]

    def kfn(x_ref, w_ref, o_ref):
        o_ref[...] = jnp.dot(x_ref[...], w_ref[...],
                             preferred_element_type=jnp.float32)

    return pl.pallas_call(
        kfn,
        grid=(M // BM,),
        in_specs=[pl.BlockSpec((BM, D), lambda i: (i, 0)),
                  pl.BlockSpec((D, D), lambda i: (0, 0))],
        out_specs=pl.BlockSpec((BM, D), lambda i: (i, 0)),
        out_shape=jax.ShapeDtypeStruct((M, D), jnp.float32),
    )(x, W)


def _comb_mm(acc, b2d, W):
    def kfn(a0_ref, a1_ref, b_ref, w_ref, o_ref):
        h = jnp.maximum(a0_ref[0] + a1_ref[0] + b_ref[...], 0.0)
        o_ref[...] = jnp.dot(h, w_ref[...],
                             preferred_element_type=jnp.float32)

    return pl.pallas_call(
        kfn,
        grid=(N_NODES // BM,),
        in_specs=[pl.BlockSpec((1, BM, D), lambda i: (0, i, 0)),
                  pl.BlockSpec((1, BM, D), lambda i: (1, i, 0)),
                  pl.BlockSpec((1, D), lambda i: (0, 0)),
                  pl.BlockSpec((D, D), lambda i: (0, 0))],
        out_specs=pl.BlockSpec((BM, D), lambda i: (i, 0)),
        out_shape=jax.ShapeDtypeStruct((N_NODES, D), jnp.float32),
    )(acc, acc, b2d, W)


def _comb(acc, b2d):
    def kfn(a0_ref, a1_ref, b_ref, o_ref):
        o_ref[...] = jnp.maximum(a0_ref[0] + a1_ref[0] + b_ref[...], 0.0)

    return pl.pallas_call(
        kfn,
        grid=(N_NODES // BM,),
        in_specs=[pl.BlockSpec((1, BM, D), lambda i: (0, i, 0)),
                  pl.BlockSpec((1, BM, D), lambda i: (1, i, 0)),
                  pl.BlockSpec((1, D), lambda i: (0, 0))],
        out_specs=pl.BlockSpec((BM, D), lambda i: (i, 0)),
        out_shape=jax.ShapeDtypeStruct((N_NODES, D), jnp.float32),
    )(acc, acc, b2d)


def kernel(x, edge_index, W1, b1, W2, b2):
    src = edge_index[0].astype(jnp.int32)
    dst = edge_index[1].astype(jnp.int32)
    n_edges = src.shape[0]
    pad = E_PAD - n_edges
    packed = jnp.bitwise_or(jnp.left_shift(dst, 16), src)
    packed3 = jnp.concatenate(
        [packed, jnp.full((pad,), PAD_DST << 16, jnp.int32)]
    ).reshape(NW, CPW, CHUNK)
    b1r = b1.reshape(1, D)
    b2r = b2.reshape(1, D)

    h1 = _mm(x, W1)
    acc1 = _seg_sum_sc(h1, packed3)
    h2 = _comb_mm(acc1, b1r, W2)
    acc2 = _seg_sum_sc(h2, packed3)
    return _comb(acc2, b2r)
